# baseline (speedup 1.0000x reference)
import functools

import jax
import jax.numpy as jnp
from jax.experimental import pallas as pl
from jax.experimental.pallas import tpu as pltpu


def _fsmn_kernel(*args, T, Tc, lorder, n_layers):
    x_ref, w_in_ref, b_in_ref = args[0:3]
    per = args[3:3 + 5 * n_layers]
    w_out_ref, b_out_ref = args[3 + 5 * n_layers:5 + 5 * n_layers]
    probs_ref = args[5 + 5 * n_layers]
    nc_refs = args[6 + 5 * n_layers:]

    Bc, _, Din = x_ref.shape
    M = T * Bc

    x2 = jnp.swapaxes(x_ref[...].astype(jnp.bfloat16), 0, 1).reshape(M, Din)
    h = jnp.dot(x2, w_in_ref[...], preferred_element_type=jnp.float32)
    h = jnp.maximum(h + b_in_ref[...], 0.0)

    for li in range(n_layers):
        lw_ref, filt_ref, aw_ref, ab_ref, c_ref = per[5 * li:5 * li + 5]
        p = jnp.dot(h.astype(jnp.bfloat16), lw_ref[...],
                    preferred_element_type=jnp.float32)
        Dp = p.shape[-1]
        p3 = p.reshape(T, Bc, Dp)
        nc_refs[li][...] = jnp.swapaxes(p3[T - Tc:].astype(jnp.bfloat16), 0, 1)
        p_bf = p3.astype(jnp.bfloat16)
        xc = jnp.concatenate([c_ref[...], p_bf], axis=0)
        half = lorder // 2
        c0 = filt_ref[0][None, None, :] * xc[0:T]
        for k in range(1, half):
            c0 = c0 + filt_ref[k][None, None, :] * xc[k:k + T]
        c1 = filt_ref[half][None, None, :] * xc[half:half + T]
        for k in range(half + 1, lorder):
            c1 = c1 + filt_ref[k][None, None, :] * xc[k:k + T]
        acc = p3 + (c0 + c1).astype(jnp.float32)
        y = jnp.dot(acc.reshape(M, Dp).astype(jnp.bfloat16), aw_ref[...],
                    preferred_element_type=jnp.float32) + ab_ref[...]
        h = jnp.maximum(y, 0.0)

    logits = jnp.dot(h.astype(jnp.bfloat16), w_out_ref[...],
                     preferred_element_type=jnp.float32) + b_out_ref[...]
    m = jnp.max(logits, axis=-1, keepdims=True)
    e = jnp.exp(logits - m)
    probs = e / jnp.sum(e, axis=-1, keepdims=True)
    Dout = probs.shape[-1]
    probs_ref[...] = jnp.swapaxes(
        probs.reshape(T, Bc, Dout), 0, 1).astype(probs_ref.dtype)


def kernel(x, in_w1, in_b1, in_w2, in_b2, out_w1, out_b1, out_w2, out_b2,
           lin_w_0, filt_0, aff_w_0, aff_b_0, cache_0,
           lin_w_1, filt_1, aff_w_1, aff_b_1, cache_1,
           lin_w_2, filt_2, aff_w_2, aff_b_2, cache_2,
           lin_w_3, filt_3, aff_w_3, aff_b_3, cache_3):
    B, T, Din = x.shape
    lin_ws = [lin_w_0, lin_w_1, lin_w_2, lin_w_3]
    filts = [filt_0, filt_1, filt_2, filt_3]
    aff_ws = [aff_w_0, aff_w_1, aff_w_2, aff_w_3]
    aff_bs = [aff_b_0, aff_b_1, aff_b_2, aff_b_3]
    caches_pt = [cache_0, cache_1, cache_2, cache_3]
    L = len(lin_ws)
    lorder = filts[0].shape[0]
    Tc = caches_pt[0].shape[2]
    Dl = in_w2.shape[1]
    Dp = lin_ws[0].shape[1]
    Dout = out_w2.shape[1]

    w_in = jnp.dot(in_w1, in_w2).astype(jnp.bfloat16)
    b_in = (jnp.dot(in_b1[None, :], in_w2) + in_b2[None, :])
    w_out = jnp.dot(out_w1, out_w2).astype(jnp.bfloat16)
    b_out = (jnp.dot(out_b1[None, :], out_w2) + out_b2[None, :])

    caches = [jnp.transpose(c[..., 0], (2, 0, 1)).astype(jnp.bfloat16)
              for c in caches_pt]

    Bc = 32
    nb = B // Bc

    in_specs = [
        pl.BlockSpec((Bc, T, Din), lambda i: (i, 0, 0)),
        pl.BlockSpec((Din, Dl), lambda i: (0, 0)),
        pl.BlockSpec((1, Dl), lambda i: (0, 0)),
    ]
    operands = [x, w_in, b_in]
    for li in range(L):
        in_specs += [
            pl.BlockSpec((Dl, Dp), lambda i: (0, 0)),
            pl.BlockSpec((lorder, Dp), lambda i: (0, 0)),
            pl.BlockSpec((Dp, Dl), lambda i: (0, 0)),
            pl.BlockSpec((1, Dl), lambda i: (0, 0)),
            pl.BlockSpec((Tc, Bc, Dp), lambda i: (0, i, 0)),
        ]
        operands += [lin_ws[li].astype(jnp.bfloat16),
                     filts[li].astype(jnp.bfloat16),
                     aff_ws[li].astype(jnp.bfloat16),
                     aff_bs[li].reshape(1, Dl), caches[li]]
    in_specs += [
        pl.BlockSpec((Dl, Dout), lambda i: (0, 0)),
        pl.BlockSpec((1, Dout), lambda i: (0, 0)),
    ]
    operands += [w_out, b_out]

    out_shape = [jax.ShapeDtypeStruct((B, T, Dout), x.dtype)]
    out_specs = [pl.BlockSpec((Bc, T, Dout), lambda i: (i, 0, 0))]
    for li in range(L):
        out_shape.append(jax.ShapeDtypeStruct((B, Tc, Dp), jnp.bfloat16))
        out_specs.append(pl.BlockSpec((Bc, Tc, Dp), lambda i: (i, 0, 0)))

    outs = pl.pallas_call(
        functools.partial(_fsmn_kernel, T=T, Tc=Tc, lorder=lorder,
                          n_layers=L),
        out_shape=tuple(out_shape),
        grid=(nb,),
        in_specs=in_specs,
        out_specs=tuple(out_specs),
        compiler_params=pltpu.CompilerParams(
            dimension_semantics=("parallel",)),
    )(*operands)

    probs = outs[0]
    out_caches = [jnp.transpose(nc, (0, 2, 1)).astype(x.dtype)[..., None]
                  for nc in outs[1:]]
    return probs, out_caches[0], out_caches[1], out_caches[2], out_caches[3]


# speedup vs baseline: 1.0370x; 1.0370x over previous
import functools

import jax
import jax.numpy as jnp
from jax.experimental import pallas as pl
from jax.experimental.pallas import tpu as pltpu


def _fsmn_kernel(*args, T, Tc, lorder, n_layers):
    x_ref, w_in_ref, b_in_ref = args[0:3]
    per = args[3:3 + 5 * n_layers]
    w_out_ref, b_out_ref = args[3 + 5 * n_layers:5 + 5 * n_layers]
    probs_ref = args[5 + 5 * n_layers]
    nc_refs = args[6 + 5 * n_layers:]

    Bc, _, Din = x_ref.shape
    M = T * Bc

    x2 = jnp.swapaxes(x_ref[...].astype(jnp.bfloat16), 0, 1).reshape(M, Din)
    h = jnp.dot(x2, w_in_ref[...], preferred_element_type=jnp.float32)
    h = jnp.maximum((h + b_in_ref[...]).astype(jnp.bfloat16), 0)

    for li in range(n_layers):
        lw_ref, filt_ref, aw_ref, ab_ref, c_ref = per[5 * li:5 * li + 5]
        p = jnp.dot(h, lw_ref[...], preferred_element_type=jnp.float32)
        Dp = p.shape[-1]
        p3 = p.reshape(T, Bc, Dp)
        nc_refs[li][...] = p3[T - Tc:].astype(jnp.bfloat16)
        p_bf = p3.astype(jnp.bfloat16)
        xc = jnp.concatenate([c_ref[...], p_bf], axis=0)
        half = lorder // 2
        c0 = filt_ref[0][None, None, :] * xc[0:T]
        for k in range(1, half):
            c0 = c0 + filt_ref[k][None, None, :] * xc[k:k + T]
        c1 = filt_ref[half][None, None, :] * xc[half:half + T]
        for k in range(half + 1, lorder):
            c1 = c1 + filt_ref[k][None, None, :] * xc[k:k + T]
        acc = p3 + (c0 + c1).astype(jnp.float32)
        y = jnp.dot(acc.reshape(M, Dp).astype(jnp.bfloat16), aw_ref[...],
                    preferred_element_type=jnp.float32) + ab_ref[...]
        h = jnp.maximum(y.astype(jnp.bfloat16), 0)

    logits = jnp.dot(h, w_out_ref[...],
                     preferred_element_type=jnp.float32) + b_out_ref[...]
    m = jnp.max(logits, axis=-1, keepdims=True)
    e = jnp.exp(logits - m)
    probs = e / jnp.sum(e, axis=-1, keepdims=True)
    Dout = probs.shape[-1]
    probs_ref[...] = jnp.swapaxes(
        probs.reshape(T, Bc, Dout), 0, 1).astype(probs_ref.dtype)


def kernel(x, in_w1, in_b1, in_w2, in_b2, out_w1, out_b1, out_w2, out_b2,
           lin_w_0, filt_0, aff_w_0, aff_b_0, cache_0,
           lin_w_1, filt_1, aff_w_1, aff_b_1, cache_1,
           lin_w_2, filt_2, aff_w_2, aff_b_2, cache_2,
           lin_w_3, filt_3, aff_w_3, aff_b_3, cache_3):
    B, T, Din = x.shape
    lin_ws = [lin_w_0, lin_w_1, lin_w_2, lin_w_3]
    filts = [filt_0, filt_1, filt_2, filt_3]
    aff_ws = [aff_w_0, aff_w_1, aff_w_2, aff_w_3]
    aff_bs = [aff_b_0, aff_b_1, aff_b_2, aff_b_3]
    caches_pt = [cache_0, cache_1, cache_2, cache_3]
    L = len(lin_ws)
    lorder = filts[0].shape[0]
    Tc = caches_pt[0].shape[2]
    Dl = in_w2.shape[1]
    Dp = lin_ws[0].shape[1]
    Dout = out_w2.shape[1]

    w_in = jnp.dot(in_w1, in_w2).astype(jnp.bfloat16)
    b_in = (jnp.dot(in_b1[None, :], in_w2) + in_b2[None, :])
    w_out = jnp.dot(out_w1, out_w2).astype(jnp.bfloat16)
    b_out = (jnp.dot(out_b1[None, :], out_w2) + out_b2[None, :])

    caches = [jnp.transpose(c[..., 0], (2, 0, 1)).astype(jnp.bfloat16)
              for c in caches_pt]

    Bc = 32
    nb = B // Bc

    in_specs = [
        pl.BlockSpec((Bc, T, Din), lambda i: (i, 0, 0)),
        pl.BlockSpec((Din, Dl), lambda i: (0, 0)),
        pl.BlockSpec((1, Dl), lambda i: (0, 0)),
    ]
    operands = [x, w_in, b_in]
    for li in range(L):
        in_specs += [
            pl.BlockSpec((Dl, Dp), lambda i: (0, 0)),
            pl.BlockSpec((lorder, Dp), lambda i: (0, 0)),
            pl.BlockSpec((Dp, Dl), lambda i: (0, 0)),
            pl.BlockSpec((1, Dl), lambda i: (0, 0)),
            pl.BlockSpec((Tc, Bc, Dp), lambda i: (0, i, 0)),
        ]
        operands += [lin_ws[li].astype(jnp.bfloat16),
                     filts[li].astype(jnp.bfloat16),
                     aff_ws[li].astype(jnp.bfloat16),
                     aff_bs[li].reshape(1, Dl), caches[li]]
    in_specs += [
        pl.BlockSpec((Dl, Dout), lambda i: (0, 0)),
        pl.BlockSpec((1, Dout), lambda i: (0, 0)),
    ]
    operands += [w_out, b_out]

    out_shape = [jax.ShapeDtypeStruct((B, T, Dout), x.dtype)]
    out_specs = [pl.BlockSpec((Bc, T, Dout), lambda i: (i, 0, 0))]
    for li in range(L):
        out_shape.append(jax.ShapeDtypeStruct((Tc, B, Dp), jnp.bfloat16))
        out_specs.append(pl.BlockSpec((Tc, Bc, Dp), lambda i: (0, i, 0)))

    outs = pl.pallas_call(
        functools.partial(_fsmn_kernel, T=T, Tc=Tc, lorder=lorder,
                          n_layers=L),
        out_shape=tuple(out_shape),
        grid=(nb,),
        in_specs=in_specs,
        out_specs=tuple(out_specs),
        compiler_params=pltpu.CompilerParams(
            dimension_semantics=("parallel",)),
    )(*operands)

    probs = outs[0]
    out_caches = [jnp.transpose(nc, (1, 2, 0)).astype(x.dtype)[..., None]
                  for nc in outs[1:]]
    return probs, out_caches[0], out_caches[1], out_caches[2], out_caches[3]
